# baseline (speedup 1.0000x reference)
import functools

import jax
import jax.numpy as jnp
from jax import lax
from jax.experimental import pallas as pl
from jax.experimental.pallas import tpu as pltpu
from jax.experimental.pallas import tpu_sc as plsc

NC, NS = 2, 16
NW = NC * NS
LANES = 16


def _round_up(n, m):
    return (n + m - 1) // m * m


def _sc_mesh():
    return plsc.VectorSubcoreMesh(
        core_axis_name="c", subcore_axis_name="s",
        num_cores=NC, num_subcores=NS)


def _make_deg_kernel(E, N):
    CH = 128
    epw = (E // CH // NW) * CH
    nextra = E // CH - (epw // CH) * NW
    ngroups = epw // LANES

    @functools.partial(
        pl.kernel,
        out_type=jax.ShapeDtypeStruct((NW * N,), jnp.float32),
        mesh=_sc_mesh(),
        compiler_params=pltpu.CompilerParams(needs_layout_passes=False),
        scratch_types=[
            pltpu.VMEM((2, epw), jnp.int32),
            pltpu.VMEM((N,), jnp.float32),
        ],
    )
    def deg_kernel(ei_hbm, out_hbm, ei_v, deg_v):
        c = lax.axis_index("c")
        s = lax.axis_index("s")
        wid = c * NS + s

        def zero_body(i, _):
            deg_v[pl.ds(i * LANES, LANES)] = jnp.zeros((LANES,), jnp.float32)
            return _

        lax.fori_loop(0, N // LANES, zero_body, None)

        pltpu.sync_copy(ei_hbm.at[:, pl.ds(wid * epw, epw)], ei_v)
        ones = jnp.ones((LANES,), jnp.float32)

        def body(g, _):
            idx = ei_v[1, pl.ds(g * LANES, LANES)]
            plsc.addupdate_scatter(deg_v, [idx], ones)
            return _

        lax.fori_loop(0, ngroups, body, None)

        if nextra:
            @pl.when(wid < nextra)
            def _():
                pltpu.sync_copy(
                    ei_hbm.at[:, pl.ds(NW * epw + wid * CH, CH)],
                    ei_v.at[:, pl.ds(0, CH)])
                lax.fori_loop(0, CH // LANES, body, None)

        pltpu.sync_copy(deg_v, out_hbm.at[pl.ds(wid * N, N)])

    return deg_kernel


def _make_agg_kernel(E, N, D):
    CH = 128
    nchunks = E // CH
    nfull = nchunks // NW
    nextra = nchunks - nfull * NW
    npad = _round_up(N, NS * 8)
    rpw = npad // NS

    @functools.partial(
        pl.kernel,
        out_type=jax.ShapeDtypeStruct((NC, npad, D), jnp.float32),
        mesh=_sc_mesh(),
        scratch_types=(
            [pltpu.VMEM((2, CH), jnp.int32)] * 3
            + [pltpu.VMEM((CH, D), jnp.float32)] * 3
            + [pltpu.VMEM_SHARED((npad, D), jnp.float32)]
            + [pltpu.SemaphoreType.DMA] * 9
        ),
    )
    def agg_kernel(ei_hbm, zeros_hbm, hs_hbm, out_hbm,
                   ib0, ib1, ib2, rb0, rb1, rb2, acc,
                   si0, si1, si2, sg0, sg1, sg2, ss0, ss1, ss2):
        idx = (ib0, ib1, ib2)
        rows = (rb0, rb1, rb2)
        sem_i = (si0, si1, si2)
        sem_g = (sg0, sg1, sg2)
        sem_s = (ss0, ss1, ss2)
        c = lax.axis_index("c")
        s = lax.axis_index("s")
        w = c * NS + s

        def ei_slice(g):
            return ei_hbm.at[:, pl.ds((g * NW + w) * CH, CH)]

        def idx_start(b, g):
            pltpu.async_copy(ei_slice(g), idx[b], sem_i[b])

        def idx_wait(b, g):
            pltpu.make_async_copy(ei_slice(g), idx[b], sem_i[b]).wait()

        H = CH // 2

        def gather_start(b):
            pltpu.async_copy(hs_hbm.at[idx[b].at[0, pl.ds(0, H)]],
                             rows[b].at[pl.ds(0, H)], sem_g[b])
            pltpu.async_copy(hs_hbm.at[idx[b].at[0, pl.ds(H, H)]],
                             rows[b].at[pl.ds(H, H)], sem_g[b])

        def gather_wait(b):
            pltpu.make_async_copy(hs_hbm.at[idx[b].at[0, pl.ds(0, H)]],
                                  rows[b].at[pl.ds(0, H)], sem_g[b]).wait()
            pltpu.make_async_copy(hs_hbm.at[idx[b].at[0, pl.ds(H, H)]],
                                  rows[b].at[pl.ds(H, H)], sem_g[b]).wait()

        def scatter_start(b):
            pltpu.async_copy(rows[b], acc.at[idx[b].at[1]], sem_s[b],
                             add=True)

        def scatter_wait(b):
            pltpu.make_async_copy(rows[b], acc.at[idx[b].at[1]],
                                  sem_s[b]).wait()

        pltpu.sync_copy(zeros_hbm, acc.at[pl.ds(s * rpw, rpw)])
        plsc.subcore_barrier()

        pltpu.sync_copy(ei_slice(0), idx[0])
        gather_start(0)
        if nfull > 1:
            idx_start(1, 1)

        def slot(g, b, b1, b2):
            @pl.when(g + 1 < nfull)
            def _():
                idx_wait(b1, g + 1)

            @pl.when(g >= 1)
            def _():
                scatter_wait(b2)

            @pl.when(g + 1 < nfull)
            def _():
                gather_start(b1)

            @pl.when(g + 2 < nfull)
            def _():
                idx_start(b2, g + 2)

            gather_wait(b)
            scatter_start(b)

        def body(i, carry):
            g = 3 * i
            slot(g, 0, 1, 2)
            slot(g + 1, 1, 2, 0)
            slot(g + 2, 2, 0, 1)
            return carry

        lax.fori_loop(0, nfull // 3, body, None)
        scatter_wait((nfull - 1) % 3)

        if nextra:
            @pl.when(w < nextra)
            def _():
                pltpu.sync_copy(ei_slice(nfull), idx[0])
                gather_start(0)
                gather_wait(0)
                scatter_start(0)
                scatter_wait(0)

        plsc.subcore_barrier()
        pltpu.sync_copy(acc.at[pl.ds(s * rpw, rpw)],
                        out_hbm.at[c, pl.ds(s * rpw, rpw)])

    return agg_kernel


def _tc1a(x_ref, w_ref, t1_ref):
    t1_ref[...] = jnp.dot(x_ref[...], w_ref[...],
                          preferred_element_type=jnp.float32)


def _tc1b(t1_ref, p_ref, hs0_ref, dis_ref):
    p = p_ref[...]
    ones = jnp.ones((p.shape[0], 1), jnp.float32)
    deg = lax.dot_general(p, ones, (((0,), (0,)), ((), ()))) + 1.0
    dis = lax.rsqrt(deg)
    dis_ref[...] = dis
    hs0_ref[...] = t1_ref[...] * dis


def _tc2(p_ref, hs0_ref, dis_ref, b_ref, out_ref):
    dis = dis_ref[...]
    n = hs0_ref.shape[0]
    agg = p_ref[0, :n] + p_ref[1, :n] + hs0_ref[...]
    h = jnp.maximum(agg * dis + b_ref[...], 0.0)
    out_ref[...] = h * dis


def _tc3(p_ref, hsc_ref, dis_ref, w_ref, b_ref, mu_ref, lv_ref):
    n = hsc_ref.shape[0]
    dout = mu_ref.shape[1]
    g = (p_ref[0, :n] + p_ref[1, :n] + hsc_ref[...]) * dis_ref[...]
    out = jnp.dot(g, w_ref[...],
                  preferred_element_type=jnp.float32) + b_ref[...]
    mu_ref[...] = out[:, :dout]
    lv_ref[...] = out[:, dout:]


def kernel(x, edge_index, W1, b1, W_mu, b_mu, W_lv, b_lv):
    N, _ = x.shape
    E = edge_index.shape[1]
    D = W1.shape[1]
    Dout = W_mu.shape[1]

    npad = _round_up(N, NS * 8)
    t1 = pl.pallas_call(
        _tc1a,
        out_shape=jax.ShapeDtypeStruct((N, D), jnp.float32),
    )(x, W1)
    deg_p = _make_deg_kernel(E, N)(edge_index).reshape(NW, N)

    hs0, dis = pl.pallas_call(
        _tc1b,
        out_shape=[
            jax.ShapeDtypeStruct((N, D), jnp.float32),
            jax.ShapeDtypeStruct((N, 1), jnp.float32),
        ],
    )(t1, deg_p)

    agg = _make_agg_kernel(E, N, D)
    zeros = jnp.zeros((npad // NS, D), jnp.float32)

    p1 = agg(edge_index, zeros, hs0)

    hsc = pl.pallas_call(
        _tc2,
        out_shape=jax.ShapeDtypeStruct((N, D), jnp.float32),
    )(p1, hs0, dis, b1[None, :])

    p2 = agg(edge_index, zeros, hsc)

    Wcat = jnp.concatenate([W_mu, W_lv], axis=1)
    bcat = jnp.concatenate([b_mu, b_lv])[None, :]
    mu, lv = pl.pallas_call(
        _tc3,
        out_shape=[
            jax.ShapeDtypeStruct((N, Dout), jnp.float32),
            jax.ShapeDtypeStruct((N, Dout), jnp.float32),
        ],
    )(p2, hsc, dis, Wcat, bcat)

    return mu, lv


# speedup vs baseline: 1.0390x; 1.0390x over previous
import functools

import jax
import jax.numpy as jnp
from jax import lax
from jax.experimental import pallas as pl
from jax.experimental.pallas import tpu as pltpu
from jax.experimental.pallas import tpu_sc as plsc

NC, NS = 2, 16
NW = NC * NS
LANES = 16


def _round_up(n, m):
    return (n + m - 1) // m * m


def _sc_mesh():
    return plsc.VectorSubcoreMesh(
        core_axis_name="c", subcore_axis_name="s",
        num_cores=NC, num_subcores=NS)


def _make_deg_kernel(E, N):
    CH = 128
    epw = (E // CH // NW) * CH
    nextra = E // CH - (epw // CH) * NW
    ngroups = epw // LANES

    @functools.partial(
        pl.kernel,
        out_type=jax.ShapeDtypeStruct((NW * N,), jnp.float32),
        mesh=_sc_mesh(),
        compiler_params=pltpu.CompilerParams(needs_layout_passes=False),
        scratch_types=[
            pltpu.VMEM((2, epw), jnp.int32),
            pltpu.VMEM((N,), jnp.float32),
        ],
    )
    def deg_kernel(ei_hbm, out_hbm, ei_v, deg_v):
        c = lax.axis_index("c")
        s = lax.axis_index("s")
        wid = c * NS + s

        def zero_body(i, _):
            deg_v[pl.ds(i * LANES, LANES)] = jnp.zeros((LANES,), jnp.float32)
            return _

        lax.fori_loop(0, N // LANES, zero_body, None)

        pltpu.sync_copy(ei_hbm.at[:, pl.ds(wid * epw, epw)], ei_v)
        ones = jnp.ones((LANES,), jnp.float32)

        def body(g, _):
            idx = ei_v[1, pl.ds(g * LANES, LANES)]
            plsc.addupdate_scatter(deg_v, [idx], ones)
            return _

        lax.fori_loop(0, ngroups, body, None)

        if nextra:
            @pl.when(wid < nextra)
            def _():
                pltpu.sync_copy(
                    ei_hbm.at[:, pl.ds(NW * epw + wid * CH, CH)],
                    ei_v.at[:, pl.ds(0, CH)])
                lax.fori_loop(0, CH // LANES, body, None)

        pltpu.sync_copy(deg_v, out_hbm.at[pl.ds(wid * N, N)])

    return deg_kernel


def _make_agg_kernel(E, N, D):
    CH = 128
    nchunks = E // CH
    nfull = nchunks // NW
    nextra = nchunks - nfull * NW
    npad = _round_up(N, NS * 8)
    rpw = npad // NS

    @functools.partial(
        pl.kernel,
        out_type=jax.ShapeDtypeStruct((NC, npad, D), jnp.float32),
        mesh=_sc_mesh(),
        scratch_types=(
            [pltpu.VMEM((2, CH), jnp.int32)] * 3
            + [pltpu.VMEM((CH, D), jnp.float32)] * 3
            + [pltpu.VMEM_SHARED((npad, D), jnp.float32)]
            + [pltpu.SemaphoreType.DMA] * 9
        ),
    )
    def agg_kernel(ei_hbm, zeros_hbm, hs_hbm, out_hbm,
                   ib0, ib1, ib2, rb0, rb1, rb2, acc,
                   si0, si1, si2, sg0, sg1, sg2, ss0, ss1, ss2):
        idx = (ib0, ib1, ib2)
        rows = (rb0, rb1, rb2)
        sem_i = (si0, si1, si2)
        sem_g = (sg0, sg1, sg2)
        sem_s = (ss0, ss1, ss2)
        c = lax.axis_index("c")
        s = lax.axis_index("s")
        w = c * NS + s

        def ei_slice(g):
            return ei_hbm.at[:, pl.ds((g * NW + w) * CH, CH)]

        def idx_start(b, g):
            pltpu.async_copy(ei_slice(g), idx[b], sem_i[b])

        def idx_wait(b, g):
            pltpu.make_async_copy(ei_slice(g), idx[b], sem_i[b]).wait()

        H = CH // 2

        def gather_start(b):
            pltpu.async_copy(hs_hbm.at[idx[b].at[0, pl.ds(0, H)]],
                             rows[b].at[pl.ds(0, H)], sem_g[b])
            pltpu.async_copy(hs_hbm.at[idx[b].at[0, pl.ds(H, H)]],
                             rows[b].at[pl.ds(H, H)], sem_g[b])

        def gather_wait(b):
            pltpu.make_async_copy(hs_hbm.at[idx[b].at[0, pl.ds(0, H)]],
                                  rows[b].at[pl.ds(0, H)], sem_g[b]).wait()
            pltpu.make_async_copy(hs_hbm.at[idx[b].at[0, pl.ds(H, H)]],
                                  rows[b].at[pl.ds(H, H)], sem_g[b]).wait()

        def scatter_start(b):
            pltpu.async_copy(rows[b], acc.at[idx[b].at[1]], sem_s[b],
                             add=True)

        def scatter_wait(b):
            pltpu.make_async_copy(rows[b], acc.at[idx[b].at[1]],
                                  sem_s[b]).wait()

        pltpu.sync_copy(zeros_hbm, acc.at[pl.ds(s * rpw, rpw)])
        plsc.subcore_barrier()

        pltpu.sync_copy(ei_slice(0), idx[0])
        gather_start(0)
        if nfull > 1:
            idx_start(1, 1)

        def slot(g, b, b1, b2):
            @pl.when(g + 1 < nfull)
            def _():
                idx_wait(b1, g + 1)

            @pl.when(g >= 1)
            def _():
                scatter_wait(b2)

            @pl.when(g + 1 < nfull)
            def _():
                gather_start(b1)

            @pl.when(g + 2 < nfull)
            def _():
                idx_start(b2, g + 2)

            gather_wait(b)
            scatter_start(b)

        def body(i, carry):
            g = 3 * i
            slot(g, 0, 1, 2)
            slot(g + 1, 1, 2, 0)
            slot(g + 2, 2, 0, 1)
            return carry

        lax.fori_loop(0, nfull // 3, body, None)
        scatter_wait((nfull - 1) % 3)

        if nextra:
            @pl.when(w < nextra)
            def _():
                pltpu.sync_copy(ei_slice(nfull), idx[0])
                gather_start(0)
                gather_wait(0)
                scatter_start(0)
                scatter_wait(0)

        plsc.subcore_barrier()
        pltpu.sync_copy(acc.at[pl.ds(s * rpw, rpw)],
                        out_hbm.at[c, pl.ds(s * rpw, rpw)])

    return agg_kernel


def _tc1(x_ref, w_ref, p_ref, hs0_ref, dis_ref):
    p = p_ref[...]
    ones = jnp.ones((p.shape[0], 1), jnp.float32)
    deg = lax.dot_general(p, ones, (((0,), (0,)), ((), ()))) + 1.0
    dis = lax.rsqrt(deg)
    dis_ref[...] = dis
    hs0_ref[...] = jnp.dot(x_ref[...], w_ref[...],
                           preferred_element_type=jnp.float32) * dis


def _tc2(p_ref, hs0_ref, dis_ref, b_ref, out_ref):
    dis = dis_ref[...]
    n = hs0_ref.shape[0]
    agg = p_ref[0, :n] + p_ref[1, :n] + hs0_ref[...]
    h = jnp.maximum(agg * dis + b_ref[...], 0.0)
    out_ref[...] = h * dis


def _tc3(p_ref, hsc_ref, dis_ref, w_ref, b_ref, mu_ref, lv_ref):
    n = hsc_ref.shape[0]
    dout = mu_ref.shape[0]
    g = (p_ref[0, :n] + p_ref[1, :n] + hsc_ref[...]) * dis_ref[...]
    out = lax.dot_general(w_ref[...], g, (((0,), (1,)), ((), ()))) + b_ref[...]
    mu_ref[...] = out[:dout, :]
    lv_ref[...] = out[dout:, :]


def kernel(x, edge_index, W1, b1, W_mu, b_mu, W_lv, b_lv):
    N, _ = x.shape
    E = edge_index.shape[1]
    D = W1.shape[1]
    Dout = W_mu.shape[1]

    npad = _round_up(N, NS * 8)
    deg_p = _make_deg_kernel(E, N)(edge_index).reshape(NW, N)

    hs0, dis = pl.pallas_call(
        _tc1,
        out_shape=[
            jax.ShapeDtypeStruct((N, D), jnp.float32),
            jax.ShapeDtypeStruct((N, 1), jnp.float32),
        ],
    )(x, W1, deg_p)

    agg = _make_agg_kernel(E, N, D)
    zeros = jnp.zeros((npad // NS, D), jnp.float32)

    p1 = agg(edge_index, zeros, hs0)

    hsc = pl.pallas_call(
        _tc2,
        out_shape=jax.ShapeDtypeStruct((N, D), jnp.float32),
    )(p1, hs0, dis, b1[None, :])

    p2 = agg(edge_index, zeros, hsc)

    Wcat = jnp.concatenate([W_mu, W_lv], axis=1)
    bcat = jnp.concatenate([b_mu, b_lv])[:, None]
    mu_t, lv_t = pl.pallas_call(
        _tc3,
        out_shape=[
            jax.ShapeDtypeStruct((Dout, N), jnp.float32),
            jax.ShapeDtypeStruct((Dout, N), jnp.float32),
        ],
    )(p2, hsc, dis, Wcat, bcat)

    return mu_t.T, lv_t.T
